# baseline (speedup 1.0000x reference)
import functools

import jax
import jax.numpy as jnp
import numpy as np
from jax import lax
from jax.experimental import pallas as pl
from jax.experimental.pallas import tpu as pltpu
from jax.experimental.pallas import tpu_sc as plsc

BINS = 100
B = 16384
H = 64
ROWSTRIDE = 128
HROWS = 104
HWORDS = HROWS * ROWSTRIDE
DUMP = BINS * ROWSTRIDE

NUM_WORKERS = 32
CHUNK = B // NUM_WORKERS
LANES = 16

_t = np.arange(1, BINS, dtype=np.float32) / np.float32(BINS)
_SIM_EDGES = np.concatenate([
    np.float32(-1.0) * (np.float32(1.0) - _t) + np.float32(1.0) * _t,
    np.array([1.0], dtype=np.float32),
]).reshape(1, BINS)


def _bin_index_kernel(e1_ref, e2_ref, d_ref, se_ref, c_ref):
    e1 = e1_ref[...]
    e2 = e2_ref[...]
    d = d_ref[...]

    n1 = jnp.sqrt(jnp.sum(e1 * e1, axis=1, keepdims=True))
    n2 = jnp.sqrt(jnp.sum(e2 * e2, axis=1, keepdims=True))
    e1n = e1 / jnp.maximum(n1, 1e-12)
    e2n = e2 / jnp.maximum(n2, 1e-12)
    sim = jnp.sum(e1n * e2n, axis=1, keepdims=True)

    dmin = jnp.min(d)
    dmax = jnp.max(d)
    k = lax.broadcasted_iota(jnp.int32, (1, BINS), 1) + 1
    t = k.astype(jnp.float32) / np.float32(BINS)
    d_edges = dmin * (1.0 - t) + dmax * t
    d_edges = jnp.where(
        lax.broadcasted_iota(jnp.int32, (1, BINS), 1) == BINS - 1, dmax, d_edges)

    s_edges = se_ref[...]

    si = jnp.sum((sim >= s_edges).astype(jnp.int32), axis=1, keepdims=True)
    di = jnp.sum((d >= d_edges).astype(jnp.int32), axis=1, keepdims=True)

    valid = (sim >= -1.0) & (si <= BINS - 1) & (di <= BINS - 1)
    c_ref[...] = jnp.where(valid, si * ROWSTRIDE + di, DUMP)


def _mi_kernel(h_ref, o_ref):
    counts = jnp.sum(h_ref[...], axis=0)
    rows = lax.broadcasted_iota(jnp.int32, (HROWS, ROWSTRIDE), 0)
    cols = lax.broadcasted_iota(jnp.int32, (HROWS, ROWSTRIDE), 1)
    rmask = rows < BINS
    cmask = cols < BINS
    counts = jnp.where(rmask & cmask, counts, 0.0)

    total = jnp.sum(counts)
    jp = counts / total
    smarg = jnp.sum(jp, axis=1, keepdims=True)
    dmarg = jnp.sum(jp, axis=0, keepdims=True)
    eps = 1e-10
    log_s = jnp.log(smarg + eps)
    log_d = jnp.log(dmarg + eps)
    terms = jp * (jnp.log(jp + eps) - log_s - log_d)
    mi = jnp.sum(terms)
    n_s = -jnp.sum(jnp.where(rows[:, :1] < BINS, log_s, 0.0))
    n_d = -jnp.sum(jnp.where(cols[:1, :] < BINS, log_d, 0.0))
    o_ref[...] = (mi / jnp.minimum(n_s, n_d)).reshape(1, 1)


def _sc_hist_kernel(c_hbm, z_hbm, out_hbm, idx_v, hist_v):
    wid = lax.axis_index("s") * 2 + lax.axis_index("c")
    base = wid * CHUNK
    pltpu.sync_copy(z_hbm, hist_v)
    pltpu.sync_copy(c_hbm.at[pl.ds(base, CHUNK)], idx_v)
    for j in range(CHUNK // LANES):
        cvec = idx_v[pl.ds(j * LANES, LANES)]
        cnts, last = plsc.scan_count(cvec)
        plsc.addupdate_scatter(
            hist_v, [cvec], cnts.astype(jnp.float32), mask=last)
    pltpu.sync_copy(hist_v, out_hbm.at[wid])


@functools.cache
def _sc_hist():
    return functools.partial(
        pl.kernel,
        out_type=jax.ShapeDtypeStruct((NUM_WORKERS, HWORDS), jnp.float32),
        mesh=plsc.VectorSubcoreMesh(core_axis_name="c", subcore_axis_name="s"),
        scratch_types=[
            pltpu.VMEM((CHUNK,), jnp.int32),
            pltpu.VMEM((HWORDS,), jnp.float32),
        ],
        compiler_params=pltpu.CompilerParams(needs_layout_passes=False),
    )(_sc_hist_kernel)


def kernel(emb1, emb2, distances):
    d2 = distances.reshape(B, 1)
    c = pl.pallas_call(
        _bin_index_kernel,
        out_shape=jax.ShapeDtypeStruct((B, 1), jnp.int32),
        compiler_params=pltpu.CompilerParams(
            vmem_limit_bytes=100 * 1024 * 1024),
    )(emb1, emb2, d2, jnp.asarray(_SIM_EDGES))

    zeros = jnp.zeros((HWORDS,), jnp.float32)
    hists = _sc_hist()(c.reshape(B), zeros)

    out = pl.pallas_call(
        _mi_kernel,
        out_shape=jax.ShapeDtypeStruct((1, 1), jnp.float32),
    )(hists.reshape(NUM_WORKERS, HROWS, ROWSTRIDE))
    return out[0, 0]


# speedup vs baseline: 1.8191x; 1.8191x over previous
import functools

import jax
import jax.numpy as jnp
import numpy as np
from jax import lax
from jax.experimental import pallas as pl
from jax.experimental.pallas import tpu as pltpu
from jax.experimental.pallas import tpu_sc as plsc

BINS = 100
B = 16384
H = 64
ROWSTRIDE = 128
HROWS = 104
DUMP = BINS * ROWSTRIDE

NUM_WORKERS = 32
CHUNK = B // NUM_WORKERS
LANES = 16

NBLK = 4
BLK = B // NBLK


def _lerp_edge(kf, lo, hi):
    t = kf / np.float32(BINS)
    e = lo * (1.0 - t) + hi * t
    return jnp.where(kf == np.float32(BINS), hi, e)


def _bin_of(x, lo, hi, scale):
    u = (x - lo) * scale
    k = jnp.clip(jnp.floor(u), 0.0, np.float32(BINS - 1))
    for _ in range(2):
        down = (x < _lerp_edge(k, lo, hi)).astype(jnp.float32)
        up = (x >= _lerp_edge(k + 1.0, lo, hi)).astype(jnp.float32)
        k = jnp.clip(k + up - down, -1.0, np.float32(BINS))
    return jnp.where(jnp.isnan(u), np.float32(BINS), k)


def _stage1_kernel(e1_ref, e2_ref, d_ref, c_ref, dot_s, n1_s, n2_s):
    i = pl.program_id(0)
    e1 = e1_ref[...]
    e2 = e2_ref[...]
    ones = jnp.full((1, H), 1.0, jnp.float32)
    dn = (((1,), (1,)), ((), ()))
    dot = lax.dot_general(ones, e1 * e2, dn, preferred_element_type=jnp.float32)
    n1 = lax.dot_general(ones, e1 * e1, dn, preferred_element_type=jnp.float32)
    n2 = lax.dot_general(ones, e2 * e2, dn, preferred_element_type=jnp.float32)
    for ib in range(NBLK):
        @pl.when(i == ib)
        def _(ib=ib):
            dot_s[:, ib * BLK:(ib + 1) * BLK] = dot
            n1_s[:, ib * BLK:(ib + 1) * BLK] = n1
            n2_s[:, ib * BLK:(ib + 1) * BLK] = n2

    @pl.when(i == NBLK - 1)
    def _():
        d = d_ref[...].reshape(1, B)
        eps = 1e-12
        den = jnp.maximum(jnp.sqrt(n1_s[...]), eps) * jnp.maximum(
            jnp.sqrt(n2_s[...]), eps)
        sim = dot_s[...] / den

        dmin = jnp.min(d)
        dmax = jnp.max(d)
        si = _bin_of(sim, np.float32(-1.0), np.float32(1.0), np.float32(BINS / 2.0))
        di = _bin_of(d, dmin, dmax, np.float32(BINS) / (dmax - dmin))
        valid = (si >= 0.0) & (si < np.float32(BINS)) & \
                (di >= 0.0) & (di < np.float32(BINS))
        c = jnp.where(
            valid,
            si.astype(jnp.int32) * ROWSTRIDE + di.astype(jnp.int32),
            DUMP)
        c_ref[...] = c.reshape(B)


def _mi_kernel(h_ref, o_ref):
    counts = jnp.sum(h_ref[...], axis=0)
    rows = lax.broadcasted_iota(jnp.int32, (HROWS, ROWSTRIDE), 0)
    cols = lax.broadcasted_iota(jnp.int32, (HROWS, ROWSTRIDE), 1)
    rmask = rows < BINS
    cmask = cols < BINS
    counts = jnp.where(rmask & cmask, counts, 0.0)

    total = jnp.sum(counts)
    jp = counts / total
    smarg = jnp.sum(jp, axis=1, keepdims=True)
    dmarg = jnp.sum(jp, axis=0, keepdims=True)
    eps = 1e-10
    log_s = jnp.log(smarg + eps)
    log_d = jnp.log(dmarg + eps)
    terms = jp * (jnp.log(jp + eps) - log_s - log_d)
    mi = jnp.sum(terms)
    n_s = -jnp.sum(jnp.where(rows[:, :1] < BINS, log_s, 0.0))
    n_d = -jnp.sum(jnp.where(cols[:1, :] < BINS, log_d, 0.0))
    o_ref[...] = (mi / jnp.minimum(n_s, n_d)).reshape(1, 1)


def _sc_hist_kernel(c_hbm, z_hbm, out_hbm, idx_v, hist_v):
    wid = lax.axis_index("s") * 2 + lax.axis_index("c")
    base = wid * CHUNK
    pltpu.sync_copy(z_hbm, hist_v)
    pltpu.sync_copy(c_hbm.at[pl.ds(base, CHUNK)], idx_v)
    for j in range(CHUNK // LANES):
        cvec = idx_v[pl.ds(j * LANES, LANES)]
        cnts, last = plsc.scan_count(cvec)
        rows = lax.shift_right_logical(cvec, 7)
        colz = lax.bitwise_and(cvec, ROWSTRIDE - 1)
        plsc.addupdate_scatter(
            hist_v, [rows, colz], cnts.astype(jnp.float32), mask=last)
    pltpu.sync_copy(hist_v, out_hbm.at[wid])


@functools.cache
def _sc_hist():
    return functools.partial(
        pl.kernel,
        out_type=jax.ShapeDtypeStruct((NUM_WORKERS, HROWS, ROWSTRIDE),
                                      jnp.float32),
        mesh=plsc.VectorSubcoreMesh(core_axis_name="c", subcore_axis_name="s"),
        scratch_types=[
            pltpu.VMEM((CHUNK,), jnp.int32),
            pltpu.VMEM((HROWS, ROWSTRIDE), jnp.float32),
        ],
        compiler_params=pltpu.CompilerParams(needs_layout_passes=False),
    )(_sc_hist_kernel)


def kernel(emb1, emb2, distances):
    c = pl.pallas_call(
        _stage1_kernel,
        grid=(NBLK,),
        in_specs=[
            pl.BlockSpec((BLK, H), lambda i: (i, 0)),
            pl.BlockSpec((BLK, H), lambda i: (i, 0)),
            pl.BlockSpec((B,), lambda i: (0,)),
        ],
        out_specs=pl.BlockSpec((B,), lambda i: (0,)),
        out_shape=jax.ShapeDtypeStruct((B,), jnp.int32),
        scratch_shapes=[
            pltpu.VMEM((1, B), jnp.float32),
            pltpu.VMEM((1, B), jnp.float32),
            pltpu.VMEM((1, B), jnp.float32),
        ],
    )(emb1, emb2, distances)

    zeros = jnp.zeros((HROWS, ROWSTRIDE), jnp.float32)
    hists = _sc_hist()(c, zeros)

    out = pl.pallas_call(
        _mi_kernel,
        out_shape=jax.ShapeDtypeStruct((1, 1), jnp.float32),
    )(hists)
    return out[0, 0]


# speedup vs baseline: 2.4808x; 1.3637x over previous
import functools

import jax
import jax.numpy as jnp
import numpy as np
from jax import lax
from jax.experimental import pallas as pl
from jax.experimental.pallas import tpu as pltpu
from jax.experimental.pallas import tpu_sc as plsc

BINS = 100
B = 16384
H = 64
ROWSTRIDE = 128
HROWS = 104
DUMP = BINS * ROWSTRIDE

NUM_WORKERS = 32
CHUNK = B // NUM_WORKERS
LANES = 16

NBLK = 4
BLK = B // NBLK


def _lerp_edge(kf, lo, hi):
    t = kf / np.float32(BINS)
    e = lo * (1.0 - t) + hi * t
    return jnp.where(kf == np.float32(BINS), hi, e)


def _bin_of(x, lo, hi, scale):
    u = (x - lo) * scale
    k = jnp.clip(jnp.floor(u), 0.0, np.float32(BINS - 1))
    for _ in range(2):
        down = (x < _lerp_edge(k, lo, hi)).astype(jnp.float32)
        up = (x >= _lerp_edge(k + 1.0, lo, hi)).astype(jnp.float32)
        k = jnp.clip(k + up - down, -1.0, np.float32(BINS))
    return jnp.where(jnp.isnan(u), np.float32(BINS), k)


def _stage1_kernel(e1_ref, e2_ref, d_ref, c_ref, dot_s, n1_s, n2_s):
    i = pl.program_id(0)
    e1 = e1_ref[...]
    e2 = e2_ref[...]
    ones = jnp.full((1, H), 1.0, jnp.float32)
    dn = (((1,), (0,)), ((), ()))
    mm = functools.partial(
        lax.dot_general, dimension_numbers=dn,
        preferred_element_type=jnp.float32,
        precision=lax.Precision.HIGHEST)
    dot = mm(ones, e1 * e2)
    n1 = mm(ones, e1 * e1)
    n2 = mm(ones, e2 * e2)
    for ib in range(NBLK):
        @pl.when(i == ib)
        def _(ib=ib):
            dot_s[:, ib * BLK:(ib + 1) * BLK] = dot
            n1_s[:, ib * BLK:(ib + 1) * BLK] = n1
            n2_s[:, ib * BLK:(ib + 1) * BLK] = n2

    @pl.when(i == NBLK - 1)
    def _():
        d = d_ref[...].reshape(1, B)
        eps = 1e-12
        den = jnp.maximum(jnp.sqrt(n1_s[...]), eps) * jnp.maximum(
            jnp.sqrt(n2_s[...]), eps)
        sim = dot_s[...] / den

        dmin = jnp.min(d)
        dmax = jnp.max(d)
        si = _bin_of(sim, np.float32(-1.0), np.float32(1.0), np.float32(BINS / 2.0))
        di = _bin_of(d, dmin, dmax, np.float32(BINS) / (dmax - dmin))
        valid = (si >= 0.0) & (si < np.float32(BINS)) & \
                (di >= 0.0) & (di < np.float32(BINS))
        c = jnp.where(
            valid,
            si.astype(jnp.int32) * ROWSTRIDE + di.astype(jnp.int32),
            DUMP)
        c_ref[...] = c.reshape(B)


def _mi_kernel(h_ref, o_ref):
    counts = jnp.sum(h_ref[...], axis=0)
    rows = lax.broadcasted_iota(jnp.int32, (HROWS, ROWSTRIDE), 0)
    cols = lax.broadcasted_iota(jnp.int32, (HROWS, ROWSTRIDE), 1)
    rmask = rows < BINS
    cmask = cols < BINS
    counts = jnp.where(rmask & cmask, counts, 0.0)

    total = jnp.sum(counts)
    jp = counts / total
    smarg = jnp.sum(jp, axis=1, keepdims=True)
    dmarg = jnp.sum(jp, axis=0, keepdims=True)
    eps = 1e-10
    log_s = jnp.log(smarg + eps)
    log_d = jnp.log(dmarg + eps)
    terms = jp * (jnp.log(jp + eps) - log_s - log_d)
    mi = jnp.sum(terms)
    n_s = -jnp.sum(jnp.where(rows[:, :1] < BINS, log_s, 0.0))
    n_d = -jnp.sum(jnp.where(cols[:1, :] < BINS, log_d, 0.0))
    o_ref[...] = (mi / jnp.minimum(n_s, n_d)).reshape(1, 1)


def _sc_hist_kernel(c_hbm, z_hbm, out_hbm, idx_v, hist_v):
    wid = lax.axis_index("s") * 2 + lax.axis_index("c")
    base = wid * CHUNK
    pltpu.sync_copy(z_hbm, hist_v)
    pltpu.sync_copy(c_hbm.at[pl.ds(base, CHUNK)], idx_v)
    for j in range(CHUNK // LANES):
        cvec = idx_v[pl.ds(j * LANES, LANES)]
        cnts, last = plsc.scan_count(cvec)
        rows = lax.shift_right_logical(cvec, 7)
        colz = lax.bitwise_and(cvec, ROWSTRIDE - 1)
        plsc.addupdate_scatter(
            hist_v, [rows, colz], cnts.astype(jnp.float32), mask=last)
    pltpu.sync_copy(hist_v, out_hbm.at[wid])


@functools.cache
def _sc_hist():
    return functools.partial(
        pl.kernel,
        out_type=jax.ShapeDtypeStruct((NUM_WORKERS, HROWS, ROWSTRIDE),
                                      jnp.float32),
        mesh=plsc.VectorSubcoreMesh(core_axis_name="c", subcore_axis_name="s"),
        scratch_types=[
            pltpu.VMEM((CHUNK,), jnp.int32),
            pltpu.VMEM((HROWS, ROWSTRIDE), jnp.float32),
        ],
        compiler_params=pltpu.CompilerParams(needs_layout_passes=False),
    )(_sc_hist_kernel)


def kernel(emb1, emb2, distances):
    c = pl.pallas_call(
        _stage1_kernel,
        grid=(NBLK,),
        in_specs=[
            pl.BlockSpec((H, BLK), lambda i: (0, i)),
            pl.BlockSpec((H, BLK), lambda i: (0, i)),
            pl.BlockSpec((B,), lambda i: (0,)),
        ],
        out_specs=pl.BlockSpec((B,), lambda i: (0,)),
        out_shape=jax.ShapeDtypeStruct((B,), jnp.int32),
        scratch_shapes=[
            pltpu.VMEM((1, B), jnp.float32),
            pltpu.VMEM((1, B), jnp.float32),
            pltpu.VMEM((1, B), jnp.float32),
        ],
    )(emb1.T, emb2.T, distances)

    zeros = jnp.zeros((HROWS, ROWSTRIDE), jnp.float32)
    hists = _sc_hist()(c, zeros)

    out = pl.pallas_call(
        _mi_kernel,
        out_shape=jax.ShapeDtypeStruct((1, 1), jnp.float32),
    )(hists)
    return out[0, 0]
